# Initial kernel scaffold; baseline (speedup 1.0000x reference)
import jax
import jax.numpy as jnp
from jax.experimental import pallas as pl


def kernel(indices, weight):
    raise NotImplementedError("write your pallas kernel here")


# speedup vs baseline: 1.8450x; 1.8450x over previous
import functools

import jax
import jax.numpy as jnp
from jax import lax
from jax.experimental import pallas as pl
from jax.experimental.pallas import tpu as pltpu
from jax.experimental.pallas import tpu_sc as plsc

_INFO = plsc.get_sparse_core_info()
_NC = _INFO.num_cores
_NS = _INFO.num_subcores
_NW = _NC * _NS
_GRP = 128


@functools.lru_cache(maxsize=None)
def _build(vocab: int, dim: int, n_groups: int, k: int):
    rows_per_w = n_groups * _GRP
    steps = n_groups // k
    mesh = plsc.VectorSubcoreMesh(core_axis_name="c", subcore_axis_name="s")

    def body(table_hbm, idx_hbm, out_hbm, idx_v, rows_v, gsem):
        wid = lax.axis_index("s") * _NC + lax.axis_index("c")
        base = wid * rows_per_w

        def step_fn(g, carry):
            pltpu.sync_copy(idx_hbm.at[wid, pl.ds(g * k, k)], idx_v)
            copies = [
                pltpu.async_copy(
                    table_hbm.at[idx_v.at[j]],
                    rows_v.at[pl.ds(j * _GRP, _GRP)],
                    gsem,
                )
                for j in range(k)
            ]
            for c in copies:
                c.wait()
            pltpu.sync_copy(
                rows_v, out_hbm.at[pl.ds(base + g * (k * _GRP), k * _GRP)]
            )
            return carry

        lax.fori_loop(0, steps, step_fn, 0)

    return pl.kernel(
        body,
        out_type=jax.ShapeDtypeStruct((_NW * rows_per_w, dim), jnp.float32),
        mesh=mesh,
        compiler_params=pltpu.CompilerParams(use_tc_tiling_on_sc=False),
        scratch_types=[
            pltpu.VMEM((k, _GRP), jnp.int32),
            pltpu.VMEM((k * _GRP, dim), jnp.float32),
            pltpu.SemaphoreType.DMA,
        ],
    )


def kernel(indices, weight):
    vocab, dim = weight.shape
    out_shape = indices.shape + (dim,)
    flat = indices.reshape(-1).astype(jnp.int32)
    total = flat.shape[0]
    assert total % (_NW * _GRP) == 0, total
    n_groups = total // (_NW * _GRP)
    k = 8
    while n_groups % k:
        k -= 1
    idx3 = flat.reshape(_NW, n_groups, _GRP)
    out = _build(vocab, dim, n_groups, k)(weight, idx3)
    return out.reshape(out_shape)


# speedup vs baseline: 1.8685x; 1.0127x over previous
import functools

import jax
import jax.numpy as jnp
from jax import lax
from jax.experimental import pallas as pl
from jax.experimental.pallas import tpu as pltpu
from jax.experimental.pallas import tpu_sc as plsc

_INFO = plsc.get_sparse_core_info()
_NC = _INFO.num_cores
_NS = _INFO.num_subcores
_NW = _NC * _NS
_GRP = 128


@functools.lru_cache(maxsize=None)
def _build(vocab: int, dim: int, n_groups: int, k: int):
    rows_per_w = n_groups * _GRP
    steps = n_groups // k
    assert steps % 2 == 0, steps
    supers = steps // 2
    chunk = k * _GRP
    mesh = plsc.VectorSubcoreMesh(core_axis_name="c", subcore_axis_name="s")

    def body(table_hbm, idx_hbm, out_hbm, idx_v, rows0, rows1, gsem0, gsem1,
             wsem0, wsem1):
        wid = lax.axis_index("s") * _NC + lax.axis_index("c")
        base = wid * rows_per_w
        bufs = ((rows0, gsem0, wsem0), (rows1, gsem1, wsem1))

        def super_fn(s, carry):
            g0 = s * 2
            pltpu.sync_copy(idx_hbm.at[wid, pl.ds(g0 * k, 2 * k)], idx_v)
            gathers = []
            for b, (rows, gsem, wsem) in enumerate(bufs):
                @pl.when(s > 0)
                def _drain(rows=rows, wsem=wsem):
                    pltpu.make_async_copy(
                        rows, out_hbm.at[pl.ds(base, chunk)], wsem
                    ).wait()

                gathers.append([
                    pltpu.async_copy(
                        table_hbm.at[idx_v.at[b * k + j]],
                        rows.at[pl.ds(j * _GRP, _GRP)],
                        gsem,
                    )
                    for j in range(k)
                ])
            for b, (rows, gsem, wsem) in enumerate(bufs):
                for c in gathers[b]:
                    c.wait()
                pltpu.make_async_copy(
                    rows,
                    out_hbm.at[pl.ds(base + (g0 + b) * chunk, chunk)],
                    wsem,
                ).start()
            return carry

        lax.fori_loop(0, supers, super_fn, 0)
        for rows, gsem, wsem in bufs:
            pltpu.make_async_copy(
                rows, out_hbm.at[pl.ds(base, chunk)], wsem
            ).wait()

    return pl.kernel(
        body,
        out_type=jax.ShapeDtypeStruct((_NW * rows_per_w, dim), jnp.float32),
        mesh=mesh,
        compiler_params=pltpu.CompilerParams(use_tc_tiling_on_sc=False),
        scratch_types=[
            pltpu.VMEM((2 * k, _GRP), jnp.int32),
            pltpu.VMEM((chunk, dim), jnp.float32),
            pltpu.VMEM((chunk, dim), jnp.float32),
            pltpu.SemaphoreType.DMA,
            pltpu.SemaphoreType.DMA,
            pltpu.SemaphoreType.DMA,
            pltpu.SemaphoreType.DMA,
        ],
    )


def kernel(indices, weight):
    vocab, dim = weight.shape
    out_shape = indices.shape + (dim,)
    flat = indices.reshape(-1).astype(jnp.int32)
    total = flat.shape[0]
    assert total % (_NW * _GRP) == 0, total
    n_groups = total // (_NW * _GRP)
    k = 5
    while n_groups % (2 * k):
        k -= 1
    idx3 = flat.reshape(_NW, n_groups, _GRP)
    out = _build(vocab, dim, n_groups, k)(weight, idx3)
    return out.reshape(out_shape)
